# Initial kernel scaffold; baseline (speedup 1.0000x reference)
import jax
import jax.numpy as jnp
from jax.experimental import pallas as pl


def kernel(data):
    raise NotImplementedError("write your pallas kernel here")


# speedup vs baseline: 1.5439x; 1.5439x over previous
import functools

import jax
import jax.numpy as jnp
from jax import lax
from jax.experimental import pallas as pl
from jax.experimental.pallas import tpu as pltpu
from jax.experimental.pallas import tpu_sc as plsc

N_IN = 500000
N_OUT = 1000000
D = 32
NC = 2
NS = 16
NW = NC * NS
ROWS_PER_W = N_OUT // NW
CHUNK = 125
NCHUNK = ROWS_PER_W // CHUNK
PAD = 128

_mesh = plsc.VectorSubcoreMesh(core_axis_name="c", subcore_axis_name="s")


@functools.partial(
    pl.kernel,
    mesh=_mesh,
    compiler_params=pltpu.CompilerParams(use_tc_tiling_on_sc=False),
    out_type=jax.ShapeDtypeStruct((N_OUT, D), jnp.float32),
    scratch_types=[
        pltpu.VMEM((PAD,), jnp.int32),
        pltpu.VMEM((PAD, D), jnp.float32),
        pltpu.SemaphoreType.DMA,
    ],
)
def _stretch(data_hbm, out_hbm, idx_v, rows_v, sem):
    wid = lax.axis_index("s") * NC + lax.axis_index("c")
    tile_base = wid * ROWS_PER_W
    lane = lax.iota(jnp.int32, 16)

    def body(c, carry):
        base = tile_base + c * CHUNK
        for k in range(PAD // 16):
            j = base + (k * 16) + lane
            t = (j + ((j >> 1) & 1)) >> 1
            idx_v[pl.ds(k * 16, 16)] = jnp.minimum(t, N_IN - 1)
        pltpu.async_copy(data_hbm.at[idx_v], rows_v, sem).wait()
        pltpu.sync_copy(rows_v.at[pl.ds(0, CHUNK)],
                        out_hbm.at[pl.ds(base, CHUNK)])
        return carry

    lax.fori_loop(0, NCHUNK, body, 0)


def kernel(data):
    return _stretch(data)


# speedup vs baseline: 2.2170x; 1.4360x over previous
import functools

import jax
import jax.numpy as jnp
from jax import lax
from jax.experimental import pallas as pl
from jax.experimental.pallas import tpu as pltpu
from jax.experimental.pallas import tpu_sc as plsc

N_IN = 500000
N_OUT = 1000000
D = 32
NC = 2
NS = 16
NW = NC * NS
CHUNK = 128
SRC = 72
NFULL = N_OUT // CHUNK
NEXTRA = NFULL % NW
NBASE = NFULL // NW
TAIL = N_OUT - NFULL * CHUNK
TAIL_BASE = NFULL * CHUNK
TAIL_W = 4

_mesh = plsc.VectorSubcoreMesh(core_axis_name="c", subcore_axis_name="s")


def _smap(r):
    return (r + ((r >> 1) & 1)) >> 1


@functools.partial(
    pl.kernel,
    mesh=_mesh,
    out_type=jax.ShapeDtypeStruct((N_OUT, D), jnp.float32),
    scratch_types=[
        pltpu.VMEM((SRC, D), jnp.float32),
        pltpu.VMEM((SRC, D), jnp.float32),
        pltpu.VMEM((CHUNK, D), jnp.float32),
        pltpu.VMEM((CHUNK, D), jnp.float32),
        pltpu.SemaphoreType.DMA,
        pltpu.SemaphoreType.DMA,
        pltpu.SemaphoreType.DMA,
        pltpu.SemaphoreType.DMA,
    ],
)
def _stretch(data_hbm, out_hbm, src0, src1, dst0, dst1, rs0, rs1, ws0, ws1):
    wid = lax.axis_index("s") * NC + lax.axis_index("c")
    count = NBASE + jnp.where(wid < NEXTRA, 1, 0)

    def cidx(i):
        return wid + i * NW

    def fire_read(src, rsem, i):
        pltpu.async_copy(data_hbm.at[pl.ds(cidx(i) * (CHUNK // 2), SRC)],
                         src, rsem)

    def wait_read(src, rsem):
        pltpu.make_async_copy(data_hbm.at[pl.ds(0, SRC)], src, rsem).wait()

    def rearrange(src, dst, nrows, cap):
        for r in range(nrows):
            s = min(_smap(r), cap)
            for h in range(0, D, 16):
                dst[r, pl.ds(h, 16)] = src[s, pl.ds(h, 16)]

    def fire_write(dst, wsem, i):
        pltpu.async_copy(dst, out_hbm.at[pl.ds(cidx(i) * CHUNK, CHUNK)], wsem)

    def wait_write(dst, wsem):
        pltpu.make_async_copy(dst, out_hbm.at[pl.ds(0, CHUNK)], wsem).wait()

    fire_read(src0, rs0, 0)
    fire_read(src1, rs1, 1)

    def step(src, dst, rsem, wsem, i, first):
        wait_read(src, rsem)

        @pl.when(jnp.logical_not(first))
        def _():
            wait_write(dst, wsem)

        rearrange(src, dst, CHUNK, SRC - 1)
        fire_write(dst, wsem, i)

        @pl.when(i + 2 < count)
        def _():
            fire_read(src, rsem, i + 2)

    def body(p, carry):
        i0, i1 = 2 * p, 2 * p + 1

        @pl.when(i0 < count)
        def _():
            step(src0, dst0, rs0, ws0, i0, p == 0)

        @pl.when(i1 < count)
        def _():
            step(src1, dst1, rs1, ws1, i1, p == 0)

        return carry

    lax.fori_loop(0, (NBASE + 2) // 2, body, 0)

    wait_write(dst0, ws0)
    wait_write(dst1, ws1)

    @pl.when(wid == TAIL_W)
    def _():
        pltpu.async_copy(data_hbm.at[pl.ds(TAIL_BASE // 2, TAIL // 2)],
                         src0.at[pl.ds(0, TAIL // 2)], rs0).wait()
        rearrange(src0, dst0, TAIL, TAIL // 2 - 1)
        pltpu.sync_copy(dst0.at[pl.ds(0, TAIL)],
                        out_hbm.at[pl.ds(TAIL_BASE, TAIL)])


def kernel(data):
    return _stretch(data)
